# baseline (speedup 1.0000x reference)
import functools

import jax
import jax.numpy as jnp
from jax import lax
from jax.experimental import pallas as pl
from jax.experimental.pallas import tpu as pltpu
from jax.experimental.pallas import tpu_sc as plsc

N = 10000
E = 160000
IN_DIM = 256
HID = 512
NB = 3

NPAD = 10240
K = 64
DEPTH = 4
MP = 163840
NB16 = MP // (16 * K)
NB32 = MP // (32 * K)
RPT = NPAD // 16

R = 1024
GRID = NPAD // R


def _prop_body(u, out, acc, isrc, idst, src_rs, dst_rs,
               bufs, sems, sub, nb, init=None, halves=2):
    if init is None:
        init = u
    gb = nb // halves
    pltpu.sync_copy(init.at[pl.ds(sub * RPT, RPT)],
                    acc.at[pl.ds(sub * RPT, RPT)])
    plsc.subcore_barrier()
    for half in range(halves):
        pltpu.sync_copy(src_rs.at[pl.ds(half * gb, gb)], isrc)
        pltpu.sync_copy(dst_rs.at[pl.ds(half * gb, gb)], idst)
        for d in range(DEPTH - 1):
            pltpu.async_copy(u.at[isrc.at[d]], bufs[d], sems[d])

        def body(g, carry):
            for b in range(DEPTH):
                j = g * DEPTH + b
                pltpu.make_async_copy(u.at[isrc.at[j]], bufs[b],
                                      sems[b]).wait()

                @pl.when(j + DEPTH - 1 < gb)
                def _():
                    bn = (b + DEPTH - 1) % DEPTH
                    pltpu.async_copy(u.at[isrc.at[j + DEPTH - 1]],
                                     bufs[bn], sems[bn])

                pltpu.sync_copy(bufs[b], acc.at[idst.at[j]], add=True)
            return carry

        lax.fori_loop(0, gb // DEPTH, body, 0)
    plsc.subcore_barrier()
    pltpu.sync_copy(acc.at[pl.ds(sub * RPT, RPT)], out.at[pl.ds(sub * RPT, RPT)])
    plsc.subcore_barrier()


def _make_prop(cb):
    passes = cb // 2
    mesh = plsc.VectorSubcoreMesh(core_axis_name="c", subcore_axis_name="s")
    out_type = [jax.ShapeDtypeStruct((NPAD, 128), jnp.float32)] * cb
    scratch = [
        pltpu.VMEM_SHARED((NPAD + 16, 128), jnp.float32),
        pltpu.VMEM((NB16 // 4, K), jnp.int32),
        pltpu.VMEM((NB16 // 4, K), jnp.int32),
    ] + [pltpu.VMEM((K, 128), jnp.float32)] * DEPTH \
      + [pltpu.SemaphoreType.DMA] * DEPTH

    @functools.partial(pl.kernel, out_type=out_type, mesh=mesh,
                       scratch_types=scratch)
    def prop(*refs):
        us = refs[:cb]
        src_r, dst_r = refs[cb], refs[cb + 1]
        outs = refs[cb + 2:2 * cb + 2]
        acc, isrc, idst = refs[2 * cb + 2:2 * cb + 5]
        bufs = refs[2 * cb + 5:2 * cb + 5 + DEPTH]
        sems = refs[2 * cb + 5 + DEPTH:2 * cb + 5 + 2 * DEPTH]
        core = lax.axis_index("c")
        sub = lax.axis_index("s")
        for sc in range(2):
            @pl.when(core == sc)
            def _(sc=sc):
                for p in range(passes):
                    _prop_body(us[sc * passes + p], outs[sc * passes + p],
                               acc, isrc, idst, src_r.at[sub], dst_r.at[sub],
                               bufs, sems, sub, NB16, halves=4)

    return prop


def _make_prop_split():
    mesh = plsc.VectorSubcoreMesh(core_axis_name="c", subcore_axis_name="s")
    out_type = [jax.ShapeDtypeStruct((NPAD, 128), jnp.float32)] * 2
    scratch = [
        pltpu.VMEM_SHARED((NPAD + 16, 128), jnp.float32),
        pltpu.VMEM((NB32 // 2, K), jnp.int32),
        pltpu.VMEM((NB32 // 2, K), jnp.int32),
    ] + [pltpu.VMEM((K, 128), jnp.float32)] * DEPTH \
      + [pltpu.SemaphoreType.DMA] * DEPTH

    @functools.partial(pl.kernel, out_type=out_type, mesh=mesh,
                       scratch_types=scratch)
    def prop(u, zinit, src_r, dst_r, out_a, out_b,
             acc, isrc, idst, *bufsems):
        bufs = bufsems[:DEPTH]
        sems = bufsems[DEPTH:]
        core = lax.axis_index("c")
        sub = lax.axis_index("s")
        w = core * 16 + sub
        for sc in range(2):
            @pl.when(core == sc)
            def _(sc=sc):
                init = u if sc == 0 else zinit
                out = (out_a, out_b)[sc]
                _prop_body(u, out, acc, isrc, idst,
                           src_r.at[w], dst_r.at[w],
                           bufs, sems, sub, NB32,
                           init=init, halves=2)

    return prop


_prop2 = _make_prop(2)
_prop4 = _make_prop(4)
_prop1 = _make_prop_split()


def _row_spec(width):
    return pl.BlockSpec((R, width), lambda i: (i, 0))


def _full_spec(a, b):
    return pl.BlockSpec((a, b), lambda i: (0, 0))


def _o128(n):
    return [jax.ShapeDtypeStruct((NPAD, 128), jnp.float32)] * n


def _p0_body(deg_ref, x_ref, dinv_ref, o0, o1):
    dv = lax.rsqrt(deg_ref[...])
    dinv_ref[...] = jnp.broadcast_to(dv, (R, 128))
    o0[...] = x_ref[:, :128] * dv
    o1[...] = x_ref[:, 128:] * dv


_p0 = pl.pallas_call(
    _p0_body,
    grid=(GRID,),
    in_specs=[pl.BlockSpec((R, 1), lambda i: (i, 0)), _row_spec(256)],
    out_specs=[_row_spec(128)] * 3,
    out_shape=_o128(3),
)


def _m1_body(s0, s1, dinv, Win, bin_, W1, h_ref, o0, o1, o2, o3):
    dv = dinv[...]
    g = jnp.concatenate([s0[...] * dv, s1[...] * dv], axis=1)
    h = jnp.maximum(
        jnp.dot(g, Win[...], preferred_element_type=jnp.float32) + bin_[...],
        0.0)
    h_ref[...] = h
    u = jnp.dot(h, W1[...], preferred_element_type=jnp.float32)
    for cbi, o in enumerate((o0, o1, o2, o3)):
        o[...] = u[:, cbi * 128:(cbi + 1) * 128] * dv


_m1 = pl.pallas_call(
    _m1_body,
    grid=(GRID,),
    in_specs=[_row_spec(128)] * 3 + [_full_spec(IN_DIM, HID),
                                     _full_spec(1, HID),
                                     _full_spec(HID, HID)],
    out_specs=[_row_spec(HID)] + [_row_spec(128)] * 4,
    out_shape=[jax.ShapeDtypeStruct((NPAD, HID), jnp.float32)] + _o128(4),
)


def _mid_body(s0, s1, s2, s3, dinv, b, W, o0, o1, o2, o3):
    dv = dinv[...]
    g = jnp.concatenate([s[...] * dv for s in (s0, s1, s2, s3)], axis=1)
    o = jnp.maximum(g + b[...], 0.0)
    u = jnp.dot(o, W[...], preferred_element_type=jnp.float32)
    for cbi, oref in enumerate((o0, o1, o2, o3)):
        oref[...] = u[:, cbi * 128:(cbi + 1) * 128] * dv


_mid = pl.pallas_call(
    _mid_body,
    grid=(GRID,),
    in_specs=[_row_spec(128)] * 5 + [_full_spec(1, HID), _full_spec(HID, HID)],
    out_specs=[_row_spec(128)] * 4,
    out_shape=_o128(4),
)


def _res_body(s0, s1, s2, s3, dinv, b, hres, W, h_ref, o0, o1, o2, o3):
    dv = dinv[...]
    g = jnp.concatenate([s[...] * dv for s in (s0, s1, s2, s3)], axis=1)
    h = jnp.maximum(g + b[...] + hres[...], 0.0)
    h_ref[...] = h
    u = jnp.dot(h, W[...], preferred_element_type=jnp.float32)
    for cbi, oref in enumerate((o0, o1, o2, o3)):
        oref[...] = u[:, cbi * 128:(cbi + 1) * 128] * dv


_res = pl.pallas_call(
    _res_body,
    grid=(GRID,),
    in_specs=[_row_spec(128)] * 5 + [_full_spec(1, HID), _row_spec(HID),
                                     _full_spec(HID, HID)],
    out_specs=[_row_spec(HID)] + [_row_spec(128)] * 4,
    out_shape=[jax.ShapeDtypeStruct((NPAD, HID), jnp.float32)] + _o128(4),
)


def _resout_body(s0, s1, s2, s3, dinv, b, hres, W, o0):
    dv = dinv[...]
    g = jnp.concatenate([s[...] * dv for s in (s0, s1, s2, s3)], axis=1)
    h = jnp.maximum(g + b[...] + hres[...], 0.0)
    u = jnp.dot(h, W[...], preferred_element_type=jnp.float32)
    o0[...] = u * dv


_resout = pl.pallas_call(
    _resout_body,
    grid=(GRID,),
    in_specs=[_row_spec(128)] * 5 + [_full_spec(1, HID), _row_spec(HID),
                                     _full_spec(HID, 128)],
    out_specs=_row_spec(128),
    out_shape=_o128(1)[0],
)


def _m8_body(sa, sb, dinv, b, o):
    o[...] = (sa[...] + sb[...]) * dinv[...] + b[...]


_m8 = pl.pallas_call(
    _m8_body,
    grid=(GRID,),
    in_specs=[_row_spec(128)] * 3 + [_full_spec(1, 128)],
    out_specs=_row_spec(128),
    out_shape=_o128(1)[0],
)


def kernel(x, edge_index, W_in, b_in, Wb1, bb1, Wb2, bb2, W_out, b_out):
    src, dst = edge_index[0], edge_index[1]
    order = jnp.argsort(src)
    dst_s = jnp.take(dst, order)
    src_s = jnp.take(src, order)
    dst_sorted = jnp.sort(dst)
    rp = jnp.searchsorted(dst_sorted, jnp.arange(N + 1, dtype=jnp.int32),
                          side="left").astype(jnp.int32)
    deg = (rp[1:] - rp[:-1] + 1).astype(jnp.float32)
    deg_p = jnp.concatenate(
        [deg, jnp.ones((NPAD - N,), jnp.float32)]).reshape(NPAD, 1)

    pad_e = MP - E
    src_p = jnp.concatenate([src_s, jnp.zeros((pad_e,), jnp.int32)])
    dst_p = jnp.concatenate([dst_s, jnp.full((pad_e,), NPAD, jnp.int32)])
    src_a = src_p.reshape(16, NB16, K)
    dst_a = dst_p.reshape(16, NB16, K)
    src_b = src_p.reshape(32, NB32, K)
    dst_b = dst_p.reshape(32, NB32, K)

    xp = jnp.concatenate([x, jnp.zeros((NPAD - N, IN_DIM), jnp.float32)])
    zeros128 = jnp.zeros((NPAD, 128), jnp.float32)
    W_out_p = jnp.concatenate(
        [W_out, jnp.zeros((HID, 128 - W_out.shape[1]), jnp.float32)], axis=1)
    b_out_p = jnp.concatenate(
        [b_out, jnp.zeros((128 - b_out.shape[0],), jnp.float32)]).reshape(1, 128)
    b_in_r = b_in.reshape(1, HID)

    dinv, x0, x1 = _p0(deg_p, xp)
    s0, s1 = _prop2(x0, x1, src_a, dst_a)
    h, *u = _m1(s0, s1, dinv, W_in, b_in_r, Wb1[0])
    for i in range(NB):
        s = _prop4(*u, src_a, dst_a)
        u = _mid(*s, dinv, bb1[i].reshape(1, HID), Wb2[i])
        s = _prop4(*u, src_a, dst_a)
        if i < NB - 1:
            h, *u = _res(*s, dinv, bb2[i].reshape(1, HID), h, Wb1[i + 1])
        else:
            t = _resout(*s, dinv, bb2[i].reshape(1, HID), h, W_out_p)
    sa, sb = _prop1(t, zeros128, src_b, dst_b)
    y = _m8(sa, sb, dinv, b_out_p)
    return y[:N, :W_out.shape[1]]


# speedup vs baseline: 1.0193x; 1.0193x over previous
import functools

import jax
import jax.numpy as jnp
from jax import lax
from jax.experimental import pallas as pl
from jax.experimental.pallas import tpu as pltpu
from jax.experimental.pallas import tpu_sc as plsc

N = 10000
E = 160000
IN_DIM = 256
HID = 512
NB = 3

NPAD = 10240
K = 128
DEPTH = 2
MP = 163840
NB16 = MP // (16 * K)
NB32 = MP // (32 * K)
RPT = NPAD // 16

R = 1024
GRID = NPAD // R


def _prop_body(u, out, acc, isrc, idst, src_rs, dst_rs,
               bufs, sems, sub, nb, init=None, halves=2):
    if init is None:
        init = u
    gb = nb // halves
    pltpu.sync_copy(init.at[pl.ds(sub * RPT, RPT)],
                    acc.at[pl.ds(sub * RPT, RPT)])
    plsc.subcore_barrier()
    for half in range(halves):
        pltpu.sync_copy(src_rs.at[pl.ds(half * gb, gb)], isrc)
        pltpu.sync_copy(dst_rs.at[pl.ds(half * gb, gb)], idst)
        for d in range(DEPTH - 1):
            pltpu.async_copy(u.at[isrc.at[d]], bufs[d], sems[d])

        def body(g, carry):
            for b in range(DEPTH):
                j = g * DEPTH + b
                pltpu.make_async_copy(u.at[isrc.at[j]], bufs[b],
                                      sems[b]).wait()

                @pl.when(j + DEPTH - 1 < gb)
                def _():
                    bn = (b + DEPTH - 1) % DEPTH
                    pltpu.async_copy(u.at[isrc.at[j + DEPTH - 1]],
                                     bufs[bn], sems[bn])

                pltpu.sync_copy(bufs[b], acc.at[idst.at[j]], add=True)
            return carry

        lax.fori_loop(0, gb // DEPTH, body, 0)
    plsc.subcore_barrier()
    pltpu.sync_copy(acc.at[pl.ds(sub * RPT, RPT)], out.at[pl.ds(sub * RPT, RPT)])
    plsc.subcore_barrier()


def _make_prop(cb):
    passes = cb // 2
    mesh = plsc.VectorSubcoreMesh(core_axis_name="c", subcore_axis_name="s")
    out_type = [jax.ShapeDtypeStruct((NPAD, 128), jnp.float32)] * cb
    scratch = [
        pltpu.VMEM_SHARED((NPAD + 16, 128), jnp.float32),
        pltpu.VMEM((NB16 // 2, K), jnp.int32),
        pltpu.VMEM((NB16 // 2, K), jnp.int32),
    ] + [pltpu.VMEM((K, 128), jnp.float32)] * DEPTH \
      + [pltpu.SemaphoreType.DMA] * DEPTH

    @functools.partial(pl.kernel, out_type=out_type, mesh=mesh,
                       scratch_types=scratch)
    def prop(*refs):
        us = refs[:cb]
        src_r, dst_r = refs[cb], refs[cb + 1]
        outs = refs[cb + 2:2 * cb + 2]
        acc, isrc, idst = refs[2 * cb + 2:2 * cb + 5]
        bufs = refs[2 * cb + 5:2 * cb + 5 + DEPTH]
        sems = refs[2 * cb + 5 + DEPTH:2 * cb + 5 + 2 * DEPTH]
        core = lax.axis_index("c")
        sub = lax.axis_index("s")
        for sc in range(2):
            @pl.when(core == sc)
            def _(sc=sc):
                for p in range(passes):
                    _prop_body(us[sc * passes + p], outs[sc * passes + p],
                               acc, isrc, idst, src_r.at[sub], dst_r.at[sub],
                               bufs, sems, sub, NB16, halves=2)

    return prop


def _make_prop_split():
    mesh = plsc.VectorSubcoreMesh(core_axis_name="c", subcore_axis_name="s")
    out_type = [jax.ShapeDtypeStruct((NPAD, 128), jnp.float32)] * 2
    scratch = [
        pltpu.VMEM_SHARED((NPAD + 16, 128), jnp.float32),
        pltpu.VMEM((NB32, K), jnp.int32),
        pltpu.VMEM((NB32, K), jnp.int32),
    ] + [pltpu.VMEM((K, 128), jnp.float32)] * DEPTH \
      + [pltpu.SemaphoreType.DMA] * DEPTH

    @functools.partial(pl.kernel, out_type=out_type, mesh=mesh,
                       scratch_types=scratch)
    def prop(u, zinit, src_r, dst_r, out_a, out_b,
             acc, isrc, idst, *bufsems):
        bufs = bufsems[:DEPTH]
        sems = bufsems[DEPTH:]
        core = lax.axis_index("c")
        sub = lax.axis_index("s")
        w = core * 16 + sub
        for sc in range(2):
            @pl.when(core == sc)
            def _(sc=sc):
                init = u if sc == 0 else zinit
                out = (out_a, out_b)[sc]
                _prop_body(u, out, acc, isrc, idst,
                           src_r.at[w], dst_r.at[w],
                           bufs, sems, sub, NB32,
                           init=init, halves=1)

    return prop


_prop2 = _make_prop(2)
_prop4 = _make_prop(4)
_prop1 = _make_prop_split()


def _row_spec(width):
    return pl.BlockSpec((R, width), lambda i: (i, 0))


def _full_spec(a, b):
    return pl.BlockSpec((a, b), lambda i: (0, 0))


def _o128(n):
    return [jax.ShapeDtypeStruct((NPAD, 128), jnp.float32)] * n


def _p0_body(deg_ref, x_ref, dinv_ref, o0, o1):
    dv = lax.rsqrt(deg_ref[...])
    dinv_ref[...] = jnp.broadcast_to(dv, (R, 128))
    o0[...] = x_ref[:, :128] * dv
    o1[...] = x_ref[:, 128:] * dv


_p0 = pl.pallas_call(
    _p0_body,
    grid=(GRID,),
    in_specs=[pl.BlockSpec((R, 1), lambda i: (i, 0)), _row_spec(256)],
    out_specs=[_row_spec(128)] * 3,
    out_shape=_o128(3),
)


def _m1_body(s0, s1, dinv, Win, bin_, W1, h_ref, o0, o1, o2, o3):
    dv = dinv[...]
    g = jnp.concatenate([s0[...] * dv, s1[...] * dv], axis=1)
    h = jnp.maximum(
        jnp.dot(g, Win[...], preferred_element_type=jnp.float32) + bin_[...],
        0.0)
    h_ref[...] = h
    u = jnp.dot(h, W1[...], preferred_element_type=jnp.float32)
    for cbi, o in enumerate((o0, o1, o2, o3)):
        o[...] = u[:, cbi * 128:(cbi + 1) * 128] * dv


_m1 = pl.pallas_call(
    _m1_body,
    grid=(GRID,),
    in_specs=[_row_spec(128)] * 3 + [_full_spec(IN_DIM, HID),
                                     _full_spec(1, HID),
                                     _full_spec(HID, HID)],
    out_specs=[_row_spec(HID)] + [_row_spec(128)] * 4,
    out_shape=[jax.ShapeDtypeStruct((NPAD, HID), jnp.float32)] + _o128(4),
)


def _mid_body(s0, s1, s2, s3, dinv, b, W, o0, o1, o2, o3):
    dv = dinv[...]
    g = jnp.concatenate([s[...] * dv for s in (s0, s1, s2, s3)], axis=1)
    o = jnp.maximum(g + b[...], 0.0)
    u = jnp.dot(o, W[...], preferred_element_type=jnp.float32)
    for cbi, oref in enumerate((o0, o1, o2, o3)):
        oref[...] = u[:, cbi * 128:(cbi + 1) * 128] * dv


_mid = pl.pallas_call(
    _mid_body,
    grid=(GRID,),
    in_specs=[_row_spec(128)] * 5 + [_full_spec(1, HID), _full_spec(HID, HID)],
    out_specs=[_row_spec(128)] * 4,
    out_shape=_o128(4),
)


def _res_body(s0, s1, s2, s3, dinv, b, hres, W, h_ref, o0, o1, o2, o3):
    dv = dinv[...]
    g = jnp.concatenate([s[...] * dv for s in (s0, s1, s2, s3)], axis=1)
    h = jnp.maximum(g + b[...] + hres[...], 0.0)
    h_ref[...] = h
    u = jnp.dot(h, W[...], preferred_element_type=jnp.float32)
    for cbi, oref in enumerate((o0, o1, o2, o3)):
        oref[...] = u[:, cbi * 128:(cbi + 1) * 128] * dv


_res = pl.pallas_call(
    _res_body,
    grid=(GRID,),
    in_specs=[_row_spec(128)] * 5 + [_full_spec(1, HID), _row_spec(HID),
                                     _full_spec(HID, HID)],
    out_specs=[_row_spec(HID)] + [_row_spec(128)] * 4,
    out_shape=[jax.ShapeDtypeStruct((NPAD, HID), jnp.float32)] + _o128(4),
)


def _resout_body(s0, s1, s2, s3, dinv, b, hres, W, o0):
    dv = dinv[...]
    g = jnp.concatenate([s[...] * dv for s in (s0, s1, s2, s3)], axis=1)
    h = jnp.maximum(g + b[...] + hres[...], 0.0)
    u = jnp.dot(h, W[...], preferred_element_type=jnp.float32)
    o0[...] = u * dv


_resout = pl.pallas_call(
    _resout_body,
    grid=(GRID,),
    in_specs=[_row_spec(128)] * 5 + [_full_spec(1, HID), _row_spec(HID),
                                     _full_spec(HID, 128)],
    out_specs=_row_spec(128),
    out_shape=_o128(1)[0],
)


def _m8_body(sa, sb, dinv, b, o):
    o[...] = (sa[...] + sb[...]) * dinv[...] + b[...]


_m8 = pl.pallas_call(
    _m8_body,
    grid=(GRID,),
    in_specs=[_row_spec(128)] * 3 + [_full_spec(1, 128)],
    out_specs=_row_spec(128),
    out_shape=_o128(1)[0],
)


def kernel(x, edge_index, W_in, b_in, Wb1, bb1, Wb2, bb2, W_out, b_out):
    src, dst = edge_index[0], edge_index[1]
    order = jnp.argsort(src)
    dst_s = jnp.take(dst, order)
    src_s = jnp.take(src, order)
    dst_sorted = jnp.sort(dst)
    rp = jnp.searchsorted(dst_sorted, jnp.arange(N + 1, dtype=jnp.int32),
                          side="left").astype(jnp.int32)
    deg = (rp[1:] - rp[:-1] + 1).astype(jnp.float32)
    deg_p = jnp.concatenate(
        [deg, jnp.ones((NPAD - N,), jnp.float32)]).reshape(NPAD, 1)

    pad_e = MP - E
    src_p = jnp.concatenate([src_s, jnp.zeros((pad_e,), jnp.int32)])
    dst_p = jnp.concatenate([dst_s, jnp.full((pad_e,), NPAD, jnp.int32)])
    src_a = src_p.reshape(16, NB16, K)
    dst_a = dst_p.reshape(16, NB16, K)
    src_b = src_p.reshape(32, NB32, K)
    dst_b = dst_p.reshape(32, NB32, K)

    xp = jnp.concatenate([x, jnp.zeros((NPAD - N, IN_DIM), jnp.float32)])
    zeros128 = jnp.zeros((NPAD, 128), jnp.float32)
    W_out_p = jnp.concatenate(
        [W_out, jnp.zeros((HID, 128 - W_out.shape[1]), jnp.float32)], axis=1)
    b_out_p = jnp.concatenate(
        [b_out, jnp.zeros((128 - b_out.shape[0],), jnp.float32)]).reshape(1, 128)
    b_in_r = b_in.reshape(1, HID)

    dinv, x0, x1 = _p0(deg_p, xp)
    s0, s1 = _prop2(x0, x1, src_a, dst_a)
    h, *u = _m1(s0, s1, dinv, W_in, b_in_r, Wb1[0])
    for i in range(NB):
        s = _prop4(*u, src_a, dst_a)
        u = _mid(*s, dinv, bb1[i].reshape(1, HID), Wb2[i])
        s = _prop4(*u, src_a, dst_a)
        if i < NB - 1:
            h, *u = _res(*s, dinv, bb2[i].reshape(1, HID), h, Wb1[i + 1])
        else:
            t = _resout(*s, dinv, bb2[i].reshape(1, HID), h, W_out_p)
    sa, sb = _prop1(t, zeros128, src_b, dst_b)
    y = _m8(sa, sb, dinv, b_out_p)
    return y[:N, :W_out.shape[1]]


# speedup vs baseline: 1.1054x; 1.0846x over previous
import functools

import jax
import jax.numpy as jnp
from jax import lax
from jax.experimental import pallas as pl
from jax.experimental.pallas import tpu as pltpu
from jax.experimental.pallas import tpu_sc as plsc

N = 10000
E = 160000
IN_DIM = 256
HID = 512
NB = 3

NPAD = 10240
K = 128
DEPTH = 2
MP = 163840
NB16 = MP // (16 * K)
NB32 = MP // (32 * K)
RPT = NPAD // 16

R = 1024
GRID = NPAD // R


def _prop_body(u, out, acc, isrc, idst, src_rs, dst_rs,
               bufs, sems, ssems, sub, nb, init=None, halves=2):
    if init is None:
        init = u
    gb = nb // halves
    pltpu.sync_copy(init.at[pl.ds(sub * RPT, RPT)],
                    acc.at[pl.ds(sub * RPT, RPT)])
    plsc.subcore_barrier()
    for half in range(halves):
        pltpu.sync_copy(src_rs.at[pl.ds(half * gb, gb)], isrc)
        pltpu.sync_copy(dst_rs.at[pl.ds(half * gb, gb)], idst)
        for d in range(DEPTH - 1):
            pltpu.async_copy(u.at[isrc.at[d]], bufs[d], sems[d])

        def body(g, carry):
            for b in range(DEPTH):
                j = g * DEPTH + b
                pltpu.make_async_copy(u.at[isrc.at[j]], bufs[b],
                                      sems[b]).wait()
                pltpu.async_copy(bufs[b], acc.at[idst.at[j]], ssems[b],
                                 add=True)
                bn = (b + 1) % DEPTH

                @pl.when(jnp.logical_and(j + 1 < gb, j + 1 >= DEPTH))
                def _():
                    pltpu.make_async_copy(bufs[bn], acc.at[idst.at[j]],
                                          ssems[bn]).wait()

                @pl.when(j + 1 < gb)
                def _():
                    pltpu.async_copy(u.at[isrc.at[j + 1]],
                                     bufs[bn], sems[bn])
            return carry

        lax.fori_loop(0, gb // DEPTH, body, 0)
        for d in range(DEPTH):
            pltpu.make_async_copy(bufs[d], acc.at[idst.at[gb - DEPTH + d]],
                                  ssems[d]).wait()
    plsc.subcore_barrier()
    pltpu.sync_copy(acc.at[pl.ds(sub * RPT, RPT)], out.at[pl.ds(sub * RPT, RPT)])
    plsc.subcore_barrier()


def _make_prop(cb):
    passes = cb // 2
    mesh = plsc.VectorSubcoreMesh(core_axis_name="c", subcore_axis_name="s")
    out_type = [jax.ShapeDtypeStruct((NPAD, 128), jnp.float32)] * cb
    scratch = [
        pltpu.VMEM_SHARED((NPAD + 16, 128), jnp.float32),
        pltpu.VMEM((NB16 // 2, K), jnp.int32),
        pltpu.VMEM((NB16 // 2, K), jnp.int32),
    ] + [pltpu.VMEM((K, 128), jnp.float32)] * DEPTH \
      + [pltpu.SemaphoreType.DMA] * (2 * DEPTH)

    @functools.partial(pl.kernel, out_type=out_type, mesh=mesh,
                       scratch_types=scratch)
    def prop(*refs):
        us = refs[:cb]
        src_r, dst_r = refs[cb], refs[cb + 1]
        outs = refs[cb + 2:2 * cb + 2]
        acc, isrc, idst = refs[2 * cb + 2:2 * cb + 5]
        bufs = refs[2 * cb + 5:2 * cb + 5 + DEPTH]
        sems = refs[2 * cb + 5 + DEPTH:2 * cb + 5 + 2 * DEPTH]
        ssems = refs[2 * cb + 5 + 2 * DEPTH:2 * cb + 5 + 3 * DEPTH]
        core = lax.axis_index("c")
        sub = lax.axis_index("s")
        for sc in range(2):
            @pl.when(core == sc)
            def _(sc=sc):
                for p in range(passes):
                    _prop_body(us[sc * passes + p], outs[sc * passes + p],
                               acc, isrc, idst, src_r.at[sub], dst_r.at[sub],
                               bufs, sems, ssems, sub, NB16, halves=2)

    return prop


def _make_prop_split():
    mesh = plsc.VectorSubcoreMesh(core_axis_name="c", subcore_axis_name="s")
    out_type = [jax.ShapeDtypeStruct((NPAD, 128), jnp.float32)] * 2
    scratch = [
        pltpu.VMEM_SHARED((NPAD + 16, 128), jnp.float32),
        pltpu.VMEM((NB32, K), jnp.int32),
        pltpu.VMEM((NB32, K), jnp.int32),
    ] + [pltpu.VMEM((K, 128), jnp.float32)] * DEPTH \
      + [pltpu.SemaphoreType.DMA] * (2 * DEPTH)

    @functools.partial(pl.kernel, out_type=out_type, mesh=mesh,
                       scratch_types=scratch)
    def prop(u, zinit, src_r, dst_r, out_a, out_b,
             acc, isrc, idst, *bufsems):
        bufs = bufsems[:DEPTH]
        sems = bufsems[DEPTH:2 * DEPTH]
        ssems = bufsems[2 * DEPTH:]
        core = lax.axis_index("c")
        sub = lax.axis_index("s")
        w = core * 16 + sub
        for sc in range(2):
            @pl.when(core == sc)
            def _(sc=sc):
                init = u if sc == 0 else zinit
                out = (out_a, out_b)[sc]
                _prop_body(u, out, acc, isrc, idst,
                           src_r.at[w], dst_r.at[w],
                           bufs, sems, ssems, sub, NB32,
                           init=init, halves=1)

    return prop


_prop2 = _make_prop(2)
_prop4 = _make_prop(4)
_prop1 = _make_prop_split()


def _row_spec(width):
    return pl.BlockSpec((R, width), lambda i: (i, 0))


def _full_spec(a, b):
    return pl.BlockSpec((a, b), lambda i: (0, 0))


def _o128(n):
    return [jax.ShapeDtypeStruct((NPAD, 128), jnp.float32)] * n


def _p0_body(deg_ref, x_ref, dinv_ref, o0, o1):
    dv = lax.rsqrt(deg_ref[...])
    dinv_ref[...] = jnp.broadcast_to(dv, (R, 128))
    o0[...] = x_ref[:, :128] * dv
    o1[...] = x_ref[:, 128:] * dv


_p0 = pl.pallas_call(
    _p0_body,
    grid=(GRID,),
    in_specs=[pl.BlockSpec((R, 1), lambda i: (i, 0)), _row_spec(256)],
    out_specs=[_row_spec(128)] * 3,
    out_shape=_o128(3),
)


def _m1_body(s0, s1, dinv, Win, bin_, W1, h_ref, o0, o1, o2, o3):
    dv = dinv[...]
    g = jnp.concatenate([s0[...] * dv, s1[...] * dv], axis=1)
    h = jnp.maximum(
        jnp.dot(g, Win[...], preferred_element_type=jnp.float32) + bin_[...],
        0.0)
    h_ref[...] = h
    u = jnp.dot(h, W1[...], preferred_element_type=jnp.float32)
    for cbi, o in enumerate((o0, o1, o2, o3)):
        o[...] = u[:, cbi * 128:(cbi + 1) * 128] * dv


_m1 = pl.pallas_call(
    _m1_body,
    grid=(GRID,),
    in_specs=[_row_spec(128)] * 3 + [_full_spec(IN_DIM, HID),
                                     _full_spec(1, HID),
                                     _full_spec(HID, HID)],
    out_specs=[_row_spec(HID)] + [_row_spec(128)] * 4,
    out_shape=[jax.ShapeDtypeStruct((NPAD, HID), jnp.float32)] + _o128(4),
)


def _mid_body(s0, s1, s2, s3, dinv, b, W, o0, o1, o2, o3):
    dv = dinv[...]
    g = jnp.concatenate([s[...] * dv for s in (s0, s1, s2, s3)], axis=1)
    o = jnp.maximum(g + b[...], 0.0)
    u = jnp.dot(o, W[...], preferred_element_type=jnp.float32)
    for cbi, oref in enumerate((o0, o1, o2, o3)):
        oref[...] = u[:, cbi * 128:(cbi + 1) * 128] * dv


_mid = pl.pallas_call(
    _mid_body,
    grid=(GRID,),
    in_specs=[_row_spec(128)] * 5 + [_full_spec(1, HID), _full_spec(HID, HID)],
    out_specs=[_row_spec(128)] * 4,
    out_shape=_o128(4),
)


def _res_body(s0, s1, s2, s3, dinv, b, hres, W, h_ref, o0, o1, o2, o3):
    dv = dinv[...]
    g = jnp.concatenate([s[...] * dv for s in (s0, s1, s2, s3)], axis=1)
    h = jnp.maximum(g + b[...] + hres[...], 0.0)
    h_ref[...] = h
    u = jnp.dot(h, W[...], preferred_element_type=jnp.float32)
    for cbi, oref in enumerate((o0, o1, o2, o3)):
        oref[...] = u[:, cbi * 128:(cbi + 1) * 128] * dv


_res = pl.pallas_call(
    _res_body,
    grid=(GRID,),
    in_specs=[_row_spec(128)] * 5 + [_full_spec(1, HID), _row_spec(HID),
                                     _full_spec(HID, HID)],
    out_specs=[_row_spec(HID)] + [_row_spec(128)] * 4,
    out_shape=[jax.ShapeDtypeStruct((NPAD, HID), jnp.float32)] + _o128(4),
)


def _resout_body(s0, s1, s2, s3, dinv, b, hres, W, o0):
    dv = dinv[...]
    g = jnp.concatenate([s[...] * dv for s in (s0, s1, s2, s3)], axis=1)
    h = jnp.maximum(g + b[...] + hres[...], 0.0)
    u = jnp.dot(h, W[...], preferred_element_type=jnp.float32)
    o0[...] = u * dv


_resout = pl.pallas_call(
    _resout_body,
    grid=(GRID,),
    in_specs=[_row_spec(128)] * 5 + [_full_spec(1, HID), _row_spec(HID),
                                     _full_spec(HID, 128)],
    out_specs=_row_spec(128),
    out_shape=_o128(1)[0],
)


def _m8_body(sa, sb, dinv, b, o):
    o[...] = (sa[...] + sb[...]) * dinv[...] + b[...]


_m8 = pl.pallas_call(
    _m8_body,
    grid=(GRID,),
    in_specs=[_row_spec(128)] * 3 + [_full_spec(1, 128)],
    out_specs=_row_spec(128),
    out_shape=_o128(1)[0],
)


def kernel(x, edge_index, W_in, b_in, Wb1, bb1, Wb2, bb2, W_out, b_out):
    src, dst = edge_index[0], edge_index[1]
    order = jnp.argsort(dst)
    dst_s = jnp.take(dst, order)
    src_s = jnp.take(src, order)
    rp = jnp.searchsorted(dst_s, jnp.arange(N + 1, dtype=jnp.int32),
                          side="left").astype(jnp.int32)
    deg = (rp[1:] - rp[:-1] + 1).astype(jnp.float32)
    deg_p = jnp.concatenate(
        [deg, jnp.ones((NPAD - N,), jnp.float32)]).reshape(NPAD, 1)

    pad_e = MP - E
    src_p = jnp.concatenate([src_s, jnp.zeros((pad_e,), jnp.int32)])
    dst_p = jnp.concatenate([dst_s, jnp.full((pad_e,), NPAD, jnp.int32)])
    src_a = src_p.reshape(16, NB16, K)
    dst_a = dst_p.reshape(16, NB16, K)
    src_b = src_p.reshape(32, NB32, K)
    dst_b = dst_p.reshape(32, NB32, K)

    xp = jnp.concatenate([x, jnp.zeros((NPAD - N, IN_DIM), jnp.float32)])
    zeros128 = jnp.zeros((NPAD, 128), jnp.float32)
    W_out_p = jnp.concatenate(
        [W_out, jnp.zeros((HID, 128 - W_out.shape[1]), jnp.float32)], axis=1)
    b_out_p = jnp.concatenate(
        [b_out, jnp.zeros((128 - b_out.shape[0],), jnp.float32)]).reshape(1, 128)
    b_in_r = b_in.reshape(1, HID)

    dinv, x0, x1 = _p0(deg_p, xp)
    s0, s1 = _prop2(x0, x1, src_a, dst_a)
    h, *u = _m1(s0, s1, dinv, W_in, b_in_r, Wb1[0])
    for i in range(NB):
        s = _prop4(*u, src_a, dst_a)
        u = _mid(*s, dinv, bb1[i].reshape(1, HID), Wb2[i])
        s = _prop4(*u, src_a, dst_a)
        if i < NB - 1:
            h, *u = _res(*s, dinv, bb2[i].reshape(1, HID), h, Wb1[i + 1])
        else:
            t = _resout(*s, dinv, bb2[i].reshape(1, HID), h, W_out_p)
    sa, sb = _prop1(t, zeros128, src_b, dst_b)
    y = _m8(sa, sb, dinv, b_out_p)
    return y[:N, :W_out.shape[1]]


# speedup vs baseline: 1.3288x; 1.2020x over previous
import functools

import jax
import jax.numpy as jnp
from jax import lax
from jax.experimental import pallas as pl
from jax.experimental.pallas import tpu as pltpu
from jax.experimental.pallas import tpu_sc as plsc

N = 10000
E = 160000
IN_DIM = 256
HID = 512
NB = 3

NPAD = 10240
K = 128
DEPTH = 2
MP = 163840
NB16 = MP // (16 * K)
NB32 = MP // (32 * K)
RPT = NPAD // 16

R = 1024
GRID = NPAD // R


def _prop_body(u, out, acc, isrc, idst, src_rs, dst_rs,
               bufs, sems, ssems, sub, nb, init=None, halves=2):
    if init is None:
        init = u
    gb = nb // halves
    pltpu.sync_copy(init.at[pl.ds(sub * RPT, RPT)],
                    acc.at[pl.ds(sub * RPT, RPT)])
    plsc.subcore_barrier()
    for half in range(halves):
        pltpu.sync_copy(src_rs.at[pl.ds(half * gb, gb)], isrc)
        pltpu.sync_copy(dst_rs.at[pl.ds(half * gb, gb)], idst)
        for d in range(DEPTH - 1):
            pltpu.async_copy(u.at[isrc.at[d]], bufs[d], sems[d])

        def body(g, carry):
            for b in range(DEPTH):
                j = g * DEPTH + b
                pltpu.make_async_copy(u.at[isrc.at[j]], bufs[b],
                                      sems[b]).wait()
                pltpu.async_copy(bufs[b], acc.at[idst.at[j]], ssems[b],
                                 add=True)
                bn = (b + 1) % DEPTH

                @pl.when(jnp.logical_and(j + 1 < gb, j + 1 >= DEPTH))
                def _():
                    pltpu.make_async_copy(bufs[bn], acc.at[idst.at[j]],
                                          ssems[bn]).wait()

                @pl.when(j + 1 < gb)
                def _():
                    pltpu.async_copy(u.at[isrc.at[j + 1]],
                                     bufs[bn], sems[bn])
            return carry

        lax.fori_loop(0, gb // DEPTH, body, 0)
        for d in range(DEPTH):
            pltpu.make_async_copy(bufs[d], acc.at[idst.at[gb - DEPTH + d]],
                                  ssems[d]).wait()
    plsc.subcore_barrier()
    pltpu.sync_copy(acc.at[pl.ds(sub * RPT, RPT)], out.at[pl.ds(sub * RPT, RPT)])
    plsc.subcore_barrier()


def _make_prop(cb):
    passes = cb // 2
    mesh = plsc.VectorSubcoreMesh(core_axis_name="c", subcore_axis_name="s")
    out_type = [jax.ShapeDtypeStruct((NPAD, 128), jnp.float32)] * cb
    scratch = [
        pltpu.VMEM_SHARED((NPAD + 16, 128), jnp.float32),
        pltpu.VMEM((NB16 // 2, K), jnp.int32),
        pltpu.VMEM((NB16 // 2, K), jnp.int32),
    ] + [pltpu.VMEM((K, 128), jnp.float32)] * DEPTH \
      + [pltpu.SemaphoreType.DMA] * (2 * DEPTH)

    @functools.partial(pl.kernel, out_type=out_type, mesh=mesh,
                       scratch_types=scratch)
    def prop(*refs):
        us = refs[:cb]
        src_r, dst_r = refs[cb], refs[cb + 1]
        outs = refs[cb + 2:2 * cb + 2]
        acc, isrc, idst = refs[2 * cb + 2:2 * cb + 5]
        bufs = refs[2 * cb + 5:2 * cb + 5 + DEPTH]
        sems = refs[2 * cb + 5 + DEPTH:2 * cb + 5 + 2 * DEPTH]
        ssems = refs[2 * cb + 5 + 2 * DEPTH:2 * cb + 5 + 3 * DEPTH]
        core = lax.axis_index("c")
        sub = lax.axis_index("s")
        for sc in range(2):
            @pl.when(core == sc)
            def _(sc=sc):
                for p in range(passes):
                    _prop_body(us[sc * passes + p], outs[sc * passes + p],
                               acc, isrc, idst, src_r.at[sub], dst_r.at[sub],
                               bufs, sems, ssems, sub, NB16, halves=2)

    return prop


def _make_prop_split():
    mesh = plsc.VectorSubcoreMesh(core_axis_name="c", subcore_axis_name="s")
    out_type = [jax.ShapeDtypeStruct((NPAD, 128), jnp.float32)] * 2
    scratch = [
        pltpu.VMEM_SHARED((NPAD + 16, 128), jnp.float32),
        pltpu.VMEM((NB32, K), jnp.int32),
        pltpu.VMEM((NB32, K), jnp.int32),
    ] + [pltpu.VMEM((K, 128), jnp.float32)] * DEPTH \
      + [pltpu.SemaphoreType.DMA] * (2 * DEPTH)

    @functools.partial(pl.kernel, out_type=out_type, mesh=mesh,
                       scratch_types=scratch)
    def prop(u, zinit, src_r, dst_r, out_a, out_b,
             acc, isrc, idst, *bufsems):
        bufs = bufsems[:DEPTH]
        sems = bufsems[DEPTH:2 * DEPTH]
        ssems = bufsems[2 * DEPTH:]
        core = lax.axis_index("c")
        sub = lax.axis_index("s")
        w = core * 16 + sub
        for sc in range(2):
            @pl.when(core == sc)
            def _(sc=sc):
                init = u if sc == 0 else zinit
                out = (out_a, out_b)[sc]
                _prop_body(u, out, acc, isrc, idst,
                           src_r.at[w], dst_r.at[w],
                           bufs, sems, ssems, sub, NB32,
                           init=init, halves=1)

    return prop


_prop2 = _make_prop(2)
_prop4 = _make_prop(4)
_prop1 = _make_prop_split()


def _row_spec(width):
    return pl.BlockSpec((R, width), lambda i: (i, 0))


def _full_spec(a, b):
    return pl.BlockSpec((a, b), lambda i: (0, 0))


def _o128(n):
    return [jax.ShapeDtypeStruct((NPAD, 128), jnp.float32)] * n


def _p0_body(deg_ref, x_ref, dinv_ref, o0, o1):
    dv = lax.rsqrt(deg_ref[...])
    dinv_ref[...] = jnp.broadcast_to(dv, (R, 128))
    o0[...] = x_ref[:, :128] * dv
    o1[...] = x_ref[:, 128:] * dv


_p0 = pl.pallas_call(
    _p0_body,
    grid=(GRID,),
    in_specs=[pl.BlockSpec((R, 1), lambda i: (i, 0)), _row_spec(256)],
    out_specs=[_row_spec(128)] * 3,
    out_shape=_o128(3),
)


def _m1_body(s0, s1, dinv, Win, bin_, W1, h_ref, o0, o1, o2, o3):
    dv = dinv[...]
    g = jnp.concatenate([s0[...] * dv, s1[...] * dv], axis=1)
    h = jnp.maximum(
        jnp.dot(g, Win[...], preferred_element_type=jnp.float32) + bin_[...],
        0.0)
    h_ref[...] = h
    u = jnp.dot(h, W1[...], preferred_element_type=jnp.float32)
    for cbi, o in enumerate((o0, o1, o2, o3)):
        o[...] = u[:, cbi * 128:(cbi + 1) * 128] * dv


_m1 = pl.pallas_call(
    _m1_body,
    grid=(GRID,),
    in_specs=[_row_spec(128)] * 3 + [_full_spec(IN_DIM, HID),
                                     _full_spec(1, HID),
                                     _full_spec(HID, HID)],
    out_specs=[_row_spec(HID)] + [_row_spec(128)] * 4,
    out_shape=[jax.ShapeDtypeStruct((NPAD, HID), jnp.float32)] + _o128(4),
)


def _mid_body(s0, s1, s2, s3, dinv, b, W, o0, o1, o2, o3):
    dv = dinv[...]
    g = jnp.concatenate([s[...] * dv for s in (s0, s1, s2, s3)], axis=1)
    o = jnp.maximum(g + b[...], 0.0)
    u = jnp.dot(o, W[...], preferred_element_type=jnp.float32)
    for cbi, oref in enumerate((o0, o1, o2, o3)):
        oref[...] = u[:, cbi * 128:(cbi + 1) * 128] * dv


_mid = pl.pallas_call(
    _mid_body,
    grid=(GRID,),
    in_specs=[_row_spec(128)] * 5 + [_full_spec(1, HID), _full_spec(HID, HID)],
    out_specs=[_row_spec(128)] * 4,
    out_shape=_o128(4),
)


def _res_body(s0, s1, s2, s3, dinv, b, hres, W, h_ref, o0, o1, o2, o3):
    dv = dinv[...]
    g = jnp.concatenate([s[...] * dv for s in (s0, s1, s2, s3)], axis=1)
    h = jnp.maximum(g + b[...] + hres[...], 0.0)
    h_ref[...] = h
    u = jnp.dot(h, W[...], preferred_element_type=jnp.float32)
    for cbi, oref in enumerate((o0, o1, o2, o3)):
        oref[...] = u[:, cbi * 128:(cbi + 1) * 128] * dv


_res = pl.pallas_call(
    _res_body,
    grid=(GRID,),
    in_specs=[_row_spec(128)] * 5 + [_full_spec(1, HID), _row_spec(HID),
                                     _full_spec(HID, HID)],
    out_specs=[_row_spec(HID)] + [_row_spec(128)] * 4,
    out_shape=[jax.ShapeDtypeStruct((NPAD, HID), jnp.float32)] + _o128(4),
)


def _resout_body(s0, s1, s2, s3, dinv, b, hres, W, o0):
    dv = dinv[...]
    g = jnp.concatenate([s[...] * dv for s in (s0, s1, s2, s3)], axis=1)
    h = jnp.maximum(g + b[...] + hres[...], 0.0)
    u = jnp.dot(h, W[...], preferred_element_type=jnp.float32)
    o0[...] = u * dv


_resout = pl.pallas_call(
    _resout_body,
    grid=(GRID,),
    in_specs=[_row_spec(128)] * 5 + [_full_spec(1, HID), _row_spec(HID),
                                     _full_spec(HID, 128)],
    out_specs=_row_spec(128),
    out_shape=_o128(1)[0],
)


def _m8_body(sa, sb, dinv, b, o):
    o[...] = (sa[...] + sb[...]) * dinv[...] + b[...]


_m8 = pl.pallas_call(
    _m8_body,
    grid=(GRID,),
    in_specs=[_row_spec(128)] * 3 + [_full_spec(1, 128)],
    out_specs=_row_spec(128),
    out_shape=_o128(1)[0],
)


def kernel(x, edge_index, W_in, b_in, Wb1, bb1, Wb2, bb2, W_out, b_out):
    src, dst = edge_index[0], edge_index[1]
    dst_s, src_s = lax.sort((dst, src), num_keys=1)
    deg = (jnp.zeros((N,), jnp.int32).at[dst].add(1) + 1).astype(jnp.float32)
    deg_p = jnp.concatenate(
        [deg, jnp.ones((NPAD - N,), jnp.float32)]).reshape(NPAD, 1)

    pad_e = MP - E
    src_p = jnp.concatenate([src_s, jnp.zeros((pad_e,), jnp.int32)])
    dst_p = jnp.concatenate([dst_s, jnp.full((pad_e,), NPAD, jnp.int32)])
    src_a = src_p.reshape(16, NB16, K)
    dst_a = dst_p.reshape(16, NB16, K)
    src_b = src_p.reshape(32, NB32, K)
    dst_b = dst_p.reshape(32, NB32, K)

    xp = jnp.concatenate([x, jnp.zeros((NPAD - N, IN_DIM), jnp.float32)])
    zeros128 = jnp.zeros((NPAD, 128), jnp.float32)
    W_out_p = jnp.concatenate(
        [W_out, jnp.zeros((HID, 128 - W_out.shape[1]), jnp.float32)], axis=1)
    b_out_p = jnp.concatenate(
        [b_out, jnp.zeros((128 - b_out.shape[0],), jnp.float32)]).reshape(1, 128)
    b_in_r = b_in.reshape(1, HID)

    dinv, x0, x1 = _p0(deg_p, xp)
    s0, s1 = _prop2(x0, x1, src_a, dst_a)
    h, *u = _m1(s0, s1, dinv, W_in, b_in_r, Wb1[0])
    for i in range(NB):
        s = _prop4(*u, src_a, dst_a)
        u = _mid(*s, dinv, bb1[i].reshape(1, HID), Wb2[i])
        s = _prop4(*u, src_a, dst_a)
        if i < NB - 1:
            h, *u = _res(*s, dinv, bb2[i].reshape(1, HID), h, Wb1[i + 1])
        else:
            t = _resout(*s, dinv, bb2[i].reshape(1, HID), h, W_out_p)
    sa, sb = _prop1(t, zeros128, src_b, dst_b)
    y = _m8(sa, sb, dinv, b_out_p)
    return y[:N, :W_out.shape[1]]
